# Initial kernel scaffold; baseline (speedup 1.0000x reference)
import functools

import jax
import jax.numpy as jnp
from jax import lax
from jax.experimental import pallas as pl
from jax.experimental.pallas import tpu as pltpu
from jax.experimental.pallas import tpu_sc as plsc

N_NODES = 10000
N_PAD = 10016
N_EDGES = 320000
D = 128
NC = 2
NS = 16
NW = NC * NS
E_PER_W = N_EDGES // NW
CHUNK = 128
N_FULL = E_PER_W // CHUNK
TAIL = E_PER_W - N_FULL * CHUNK
ROWS_PER_TILE = N_PAD // NS


def _make_sc_aggregate(with_deg: bool):
    mesh = plsc.VectorSubcoreMesh(core_axis_name="c", subcore_axis_name="s")

    out_type = [jax.ShapeDtypeStruct((NC * N_PAD, D), jnp.float32)]
    scratch = [
        pltpu.VMEM((CHUNK,), jnp.int32),
        pltpu.VMEM((CHUNK,), jnp.int32),
        pltpu.VMEM((TAIL,), jnp.int32),
        pltpu.VMEM((TAIL,), jnp.int32),
        pltpu.VMEM((CHUNK, D), jnp.float32),
        pltpu.VMEM_SHARED((N_PAD, D), jnp.float32),
        pltpu.SemaphoreType.DMA,
    ]
    if with_deg:
        out_type.append(jax.ShapeDtypeStruct((NC * N_PAD, 16), jnp.float32))
        scratch += [
            pltpu.VMEM((CHUNK, 16), jnp.float32),
            pltpu.VMEM_SHARED((N_PAD, 16), jnp.float32),
        ]

    @functools.partial(
        pl.kernel, out_type=tuple(out_type), mesh=mesh,
        scratch_types=tuple(scratch),
    )
    def sc_agg(h_hbm, src_hbm, dst_hbm, z_hbm, zd_hbm, ones_hbm,
               *out_and_scratch):
        if with_deg:
            (out_hbm, outdeg_hbm, src_v, dst_v, src_t, dst_t, rows_v,
             acc, sem, ones_v, accd) = out_and_scratch
        else:
            (out_hbm, src_v, dst_v, src_t, dst_t, rows_v,
             acc, sem) = out_and_scratch
        cid = lax.axis_index("c")
        sid = lax.axis_index("s")
        wid = cid * NS + sid
        ebase = wid * E_PER_W

        r0 = sid * ROWS_PER_TILE
        pltpu.sync_copy(z_hbm.at[pl.ds(r0, ROWS_PER_TILE)],
                        acc.at[pl.ds(r0, ROWS_PER_TILE)])
        if with_deg:
            pltpu.sync_copy(zd_hbm.at[pl.ds(r0, ROWS_PER_TILE)],
                            accd.at[pl.ds(r0, ROWS_PER_TILE)])
            pltpu.sync_copy(ones_hbm, ones_v)
        plsc.subcore_barrier()

        def body(i, carry):
            off = ebase + i * CHUNK
            pltpu.sync_copy(src_hbm.at[pl.ds(off, CHUNK)], src_v)
            pltpu.sync_copy(dst_hbm.at[pl.ds(off, CHUNK)], dst_v)
            pltpu.async_copy(h_hbm.at[src_v], rows_v, sem).wait()
            pltpu.sync_copy(rows_v, acc.at[dst_v], add=True)
            if with_deg:
                pltpu.sync_copy(ones_v, accd.at[dst_v], add=True)
            return carry

        lax.fori_loop(0, N_FULL, body, 0)

        toff = ebase + N_FULL * CHUNK
        pltpu.sync_copy(src_hbm.at[pl.ds(toff, TAIL)], src_t)
        pltpu.sync_copy(dst_hbm.at[pl.ds(toff, TAIL)], dst_t)
        pltpu.async_copy(h_hbm.at[src_t], rows_v.at[pl.ds(0, TAIL)],
                         sem).wait()
        pltpu.sync_copy(rows_v.at[pl.ds(0, TAIL)], acc.at[dst_t], add=True)
        if with_deg:
            pltpu.sync_copy(ones_v.at[pl.ds(0, TAIL)], accd.at[dst_t],
                            add=True)

        plsc.subcore_barrier()

        obase = cid * N_PAD + r0
        pltpu.sync_copy(acc.at[pl.ds(r0, ROWS_PER_TILE)],
                        out_hbm.at[pl.ds(obase, ROWS_PER_TILE)])
        if with_deg:
            pltpu.sync_copy(accd.at[pl.ds(r0, ROWS_PER_TILE)],
                            outdeg_hbm.at[pl.ds(obase, ROWS_PER_TILE)])

    return sc_agg


_sc_agg_deg = _make_sc_aggregate(True)
_sc_agg = _make_sc_aggregate(False)

ROW_BLK = 1252


def _linear_body(relu, h_ref, p0_ref, p1_ref, d0_ref, d1_ref,
                 wt_ref, wb_ref, b_ref, o_ref):
    deg = jnp.maximum(d0_ref[:, :1] + d1_ref[:, :1], 1.0)
    agg = (p0_ref[...] + p1_ref[...]) / deg
    acc = jnp.dot(h_ref[...], wt_ref[...], preferred_element_type=jnp.float32)
    acc = acc + jnp.dot(agg, wb_ref[...], preferred_element_type=jnp.float32)
    acc = acc + b_ref[...]
    if relu:
        acc = jnp.maximum(acc, 0.0)
    o_ref[...] = acc


def _tc_linear(h, p0, p1, d0, d1, wt, wb, b, relu):
    grid = (N_PAD // ROW_BLK,)
    blk = lambda r, c: pl.BlockSpec((r, c), lambda i: (i, 0))
    full = lambda r, c: pl.BlockSpec((r, c), lambda i: (0, 0))
    return pl.pallas_call(
        functools.partial(_linear_body, relu),
        grid=grid,
        in_specs=[blk(ROW_BLK, D), blk(ROW_BLK, D), blk(ROW_BLK, D),
                  blk(ROW_BLK, 16), blk(ROW_BLK, 16),
                  full(D, D), full(D, D), full(1, D)],
        out_specs=blk(ROW_BLK, D),
        out_shape=jax.ShapeDtypeStruct((N_PAD, D), jnp.float32),
    )(h, p0, p1, d0, d1, wt, wb, b)


def kernel(x, edge_index, W1, b1, W2, b2, W3, b3):
    e = edge_index.astype(jnp.int32)
    src, dst = e[0], e[1]
    h = jnp.pad(x, ((0, N_PAD - N_NODES), (0, 0)))
    zeros = jnp.zeros((N_PAD, D), jnp.float32)
    zerosd = jnp.zeros((N_PAD, 16), jnp.float32)
    ones = jnp.ones((CHUNK, 16), jnp.float32)

    part, degp = _sc_agg_deg(h, src, dst, zeros, zerosd, ones)
    p0, p1 = part[:N_PAD], part[N_PAD:]
    d0, d1 = degp[:N_PAD], degp[N_PAD:]

    def agg_layer(hh):
        pt = _sc_agg(hh, src, dst, zeros, zerosd, ones)
        return pt[:N_PAD], pt[N_PAD:]

    h1 = _tc_linear(h, p0, p1, d0, d1, W1[:D], W1[D:], b1[None, :], True)
    a0, a1 = agg_layer(h1)
    h2 = _tc_linear(h1, a0, a1, d0, d1, W2[:D], W2[D:], b2[None, :], True)
    a0, a1 = agg_layer(h2)
    out = _tc_linear(h2, a0, a1, d0, d1, W3[:D], W3[D:], b3[None, :], False)
    return out[:N_NODES]


# speedup vs baseline: 6.3659x; 6.3659x over previous
import functools

import jax
import jax.numpy as jnp
from jax import lax
from jax.experimental import pallas as pl
from jax.experimental.pallas import tpu as pltpu
from jax.experimental.pallas import tpu_sc as plsc

N_NODES = 10000
N_PAD = 10112
N_EDGES = 320000
D = 128
NC = 2
NS = 16
NW = NC * NS
E_PER_W = N_EDGES // NW
CHUNK = 128
N_FULL = E_PER_W // CHUNK
TAIL = E_PER_W - N_FULL * CHUNK
ROWS_PER_TILE = N_PAD // NS
DR = N_PAD // 16

_MESH = dict(core_axis_name="c", subcore_axis_name="s")


def _make_sc_aggregate():
    mesh = plsc.VectorSubcoreMesh(**_MESH)

    @functools.partial(
        pl.kernel,
        out_type=jax.ShapeDtypeStruct((NC * N_PAD, D), jnp.float32),
        mesh=mesh,
        scratch_types=(
            pltpu.VMEM((CHUNK,), jnp.int32),
            pltpu.VMEM((CHUNK,), jnp.int32),
            pltpu.VMEM((TAIL,), jnp.int32),
            pltpu.VMEM((TAIL,), jnp.int32),
            pltpu.VMEM((CHUNK, D), jnp.float32),
            pltpu.VMEM_SHARED((N_PAD, D), jnp.float32),
            pltpu.SemaphoreType.DMA,
        ),
    )
    def sc_agg(h_hbm, src_hbm, dst_hbm, z_hbm, out_hbm,
               src_v, dst_v, src_t, dst_t, rows_v, acc, sem):
        cid = lax.axis_index("c")
        sid = lax.axis_index("s")
        wid = cid * NS + sid
        ebase = wid * E_PER_W

        r0 = sid * ROWS_PER_TILE
        pltpu.sync_copy(z_hbm.at[pl.ds(r0, ROWS_PER_TILE)],
                        acc.at[pl.ds(r0, ROWS_PER_TILE)])
        plsc.subcore_barrier()

        def body(i, carry):
            off = ebase + i * CHUNK
            pltpu.sync_copy(src_hbm.at[pl.ds(off, CHUNK)], src_v)
            pltpu.sync_copy(dst_hbm.at[pl.ds(off, CHUNK)], dst_v)
            pltpu.async_copy(h_hbm.at[src_v], rows_v, sem).wait()
            pltpu.sync_copy(rows_v, acc.at[dst_v], add=True)
            return carry

        lax.fori_loop(0, N_FULL, body, 0)

        toff = ebase + N_FULL * CHUNK
        pltpu.sync_copy(src_hbm.at[pl.ds(toff, TAIL)], src_t)
        pltpu.sync_copy(dst_hbm.at[pl.ds(toff, TAIL)], dst_t)
        pltpu.async_copy(h_hbm.at[src_t], rows_v.at[pl.ds(0, TAIL)],
                         sem).wait()
        pltpu.sync_copy(rows_v.at[pl.ds(0, TAIL)], acc.at[dst_t], add=True)

        plsc.subcore_barrier()

        obase = cid * N_PAD + r0
        pltpu.sync_copy(acc.at[pl.ds(r0, ROWS_PER_TILE)],
                        out_hbm.at[pl.ds(obase, ROWS_PER_TILE)])

    return sc_agg


def _make_sc_deg():
    mesh = plsc.VectorSubcoreMesh(**_MESH)

    @functools.partial(
        pl.kernel,
        out_type=jax.ShapeDtypeStruct((NW * DR, 16), jnp.float32),
        mesh=mesh,
        compiler_params=pltpu.CompilerParams(needs_layout_passes=False),
        scratch_types=(
            pltpu.VMEM((E_PER_W,), jnp.int32),
            pltpu.VMEM((DR, 16), jnp.float32),
        ),
    )
    def sc_deg(dst_hbm, z_hbm, out_hbm, dst_v, deg_v):
        cid = lax.axis_index("c")
        sid = lax.axis_index("s")
        wid = cid * NS + sid
        pltpu.sync_copy(dst_hbm.at[pl.ds(wid * E_PER_W, E_PER_W)], dst_v)
        pltpu.sync_copy(z_hbm, deg_v)
        ones16 = jnp.ones((16,), jnp.float32)

        def body(j, carry):
            d = dst_v[pl.ds(j * 16, 16)]
            plsc.addupdate_scatter(deg_v, [d >> 4, d & 15], ones16)
            return carry

        lax.fori_loop(0, E_PER_W // 16, body, 0)
        pltpu.sync_copy(deg_v, out_hbm.at[pl.ds(wid * DR, DR)])

    return sc_deg


_sc_agg = _make_sc_aggregate()
_sc_deg = _make_sc_deg()

ROW_BLK = 2528


def _deg_reduce_body(d_ref, o_ref):
    deg = jnp.sum(d_ref[...], axis=0)
    o_ref[...] = (1.0 / jnp.maximum(deg, 1.0))[:, None]


def _deg_reduce(degp):
    return pl.pallas_call(
        _deg_reduce_body,
        out_shape=jax.ShapeDtypeStruct((N_PAD, 1), jnp.float32),
    )(degp)


def _linear_body(relu, h_ref, p0_ref, p1_ref, di_ref, wt_ref, wb_ref,
                 b_ref, o_ref):
    agg = (p0_ref[...] + p1_ref[...]) * di_ref[...]
    acc = jnp.dot(h_ref[...], wt_ref[...], preferred_element_type=jnp.float32)
    acc = acc + jnp.dot(agg, wb_ref[...], preferred_element_type=jnp.float32)
    acc = acc + b_ref[...]
    if relu:
        acc = jnp.maximum(acc, 0.0)
    o_ref[...] = acc


def _tc_linear(h, p0, p1, dinv, wt, wb, b, relu):
    grid = (N_PAD // ROW_BLK,)
    blk = lambda r, c: pl.BlockSpec((r, c), lambda i: (i, 0))
    full = lambda r, c: pl.BlockSpec((r, c), lambda i: (0, 0))
    return pl.pallas_call(
        functools.partial(_linear_body, relu),
        grid=grid,
        in_specs=[blk(ROW_BLK, D), blk(ROW_BLK, D), blk(ROW_BLK, D),
                  blk(ROW_BLK, 1),
                  full(D, D), full(D, D), full(1, D)],
        out_specs=blk(ROW_BLK, D),
        out_shape=jax.ShapeDtypeStruct((N_PAD, D), jnp.float32),
    )(h, p0, p1, dinv, wt, wb, b)


def kernel(x, edge_index, W1, b1, W2, b2, W3, b3):
    e = edge_index.astype(jnp.int32)
    src, dst = e[0], e[1]
    h = jnp.pad(x, ((0, N_PAD - N_NODES), (0, 0)))
    zeros = jnp.zeros((N_PAD, D), jnp.float32)
    zerosd = jnp.zeros((DR, 16), jnp.float32)

    degp = _sc_deg(dst, zerosd)
    dinv = _deg_reduce(degp.reshape(NW, N_PAD))

    def agg_layer(hh):
        pt = _sc_agg(hh, src, dst, zeros)
        return pt[:N_PAD], pt[N_PAD:]

    p0, p1 = agg_layer(h)
    h1 = _tc_linear(h, p0, p1, dinv, W1[:D], W1[D:], b1[None, :], True)
    a0, a1 = agg_layer(h1)
    h2 = _tc_linear(h1, a0, a1, dinv, W2[:D], W2[D:], b2[None, :], True)
    a0, a1 = agg_layer(h2)
    out = _tc_linear(h2, a0, a1, dinv, W3[:D], W3[D:], b3[None, :], False)
    return out[:N_NODES]
